# baseline (speedup 1.0000x reference)
import functools

import jax
import jax.numpy as jnp
from jax import lax
from jax.experimental import pallas as pl
from jax.experimental.pallas import tpu as pltpu
from jax.experimental.pallas import tpu_sc as plsc

HIDDEN = 1024
BATCH = 4
SEQ = 8192
NC, NS = 2, 16
NW = NC * NS
GR = 16
NBUF = 2
JCH = HIDDEN // 16

SC_BATCH = 2
TC_BLOCK = 2048

_mesh = plsc.VectorSubcoreMesh(core_axis_name="c", subcore_axis_name="s")

_DIMS = lax.GatherDimensionNumbers(
    offset_dims=(), collapsed_slice_dims=(0,), start_index_map=(0,)
)


def _lane_splat(vec, lane):
    return lax.gather(
        vec,
        jnp.full((16, 1), lane, jnp.int32),
        _DIMS,
        (1,),
        mode=lax.GatherScatterMode.PROMISE_IN_BOUNDS,
    )


def _make_sc(batch):
    rpw = batch * SEQ // NW
    wpb = SEQ // rpw
    groups = rpw // GR

    @functools.partial(
        pl.kernel,
        mesh=_mesh,
        out_type=jax.ShapeDtypeStruct((batch, SEQ, HIDDEN), jnp.float32),
        scratch_types=[
            pltpu.VMEM((rpw,), jnp.int32),
            pltpu.VMEM((2, HIDDEN), jnp.float32),
            pltpu.VMEM((NBUF, GR, HIDDEN), jnp.float32),
            pltpu.SemaphoreType.DMA,
        ],
    )
    def _sc_lookup(seg_hbm, table_hbm, out_hbm, idx_v, tab_v, bufs, ssem):
        wid = lax.axis_index("s") * NC + lax.axis_index("c")
        bi = lax.div(wid, wpb)
        srow = lax.rem(wid, wpb) * rpw
        pltpu.sync_copy(seg_hbm.at[bi].at[pl.ds(srow, rpw)], idx_v)
        pltpu.sync_copy(table_hbm, tab_v)
        out_w = out_hbm.at[bi]

        def wait_one_scatter():
            pltpu.make_async_copy(
                out_w.at[pl.ds(srow, GR)], bufs.at[0], ssem
            ).wait()

        def outer(o, carry):
            for b in range(NBUF):
                g = o * NBUF + b
                off = pl.multiple_of(g * GR, GR)
                idx16 = idx_v[pl.ds(off, 16)]
                mults = [
                    _lane_splat(idx16, r).astype(jnp.float32)
                    for r in range(GR)
                ]

                @pl.when(o > 0)
                def _():
                    wait_one_scatter()

                def jbody(j, c, _b=b, _mults=mults):
                    jo = pl.multiple_of(j * 16, 16)
                    t0 = tab_v.at[0][pl.ds(jo, 16)]
                    d = tab_v.at[1][pl.ds(jo, 16)] - t0
                    for r in range(GR):
                        bufs.at[_b].at[r][pl.ds(jo, 16)] = t0 + _mults[r] * d
                    return c

                lax.fori_loop(0, JCH, jbody, 0)
                pltpu.async_copy(
                    bufs.at[b], out_w.at[pl.ds(srow + off, GR)], ssem
                )
            return carry

        lax.fori_loop(0, groups // NBUF, outer, 0)
        for _ in range(NBUF):
            wait_one_scatter()

    return _sc_lookup


_sc_half = _make_sc(SC_BATCH)


def _tc_body(seg_ref, tab_ref, out_ref):
    seg = seg_ref[...]
    t0 = tab_ref[0:1, :]
    t1 = tab_ref[1:2, :]
    out_ref[...] = jnp.where(seg == 0, t0, t1)


def _tc_half(seg, table):
    rows = seg.size
    seg2 = seg.reshape(rows, 1)
    return pl.pallas_call(
        _tc_body,
        grid=(rows // TC_BLOCK,),
        in_specs=[
            pl.BlockSpec((TC_BLOCK, 1), lambda i: (i, 0)),
            pl.BlockSpec((2, HIDDEN), lambda i: (0, 0)),
        ],
        out_specs=pl.BlockSpec((TC_BLOCK, HIDDEN), lambda i: (i, 0)),
        out_shape=jax.ShapeDtypeStruct((rows, HIDDEN), jnp.float32),
    )(seg2, table)


def kernel(segments, table):
    seg = segments.astype(jnp.int32)
    sc_out = _sc_half(seg[:SC_BATCH], table)
    tc_out = _tc_half(seg[SC_BATCH:], table)
    tc3 = tc_out.reshape(BATCH - SC_BATCH, SEQ, HIDDEN)
    return jnp.concatenate([sc_out, tc3], axis=0)


# speedup vs baseline: 1.3755x; 1.3755x over previous
import functools

import jax
import jax.numpy as jnp
from jax import lax
from jax.experimental import pallas as pl
from jax.experimental.pallas import tpu as pltpu
from jax.experimental.pallas import tpu_sc as plsc

HIDDEN = 1024
BATCH = 4
SEQ = 8192
ROWS = BATCH * SEQ
NC, NS = 2, 16
NW = NC * NS
RPW = ROWS // NW
WPB = SEQ // RPW
GR = 16
GROUPS = RPW // GR
NBUF = 4
SUPER = GROUPS // NBUF
JCH = HIDDEN // 16

_mesh = plsc.VectorSubcoreMesh(core_axis_name="c", subcore_axis_name="s")

_DIMS = lax.GatherDimensionNumbers(
    offset_dims=(), collapsed_slice_dims=(0,), start_index_map=(0,)
)


def _lane_splat(vec, lane):
    return lax.gather(
        vec,
        jnp.full((16, 1), lane, jnp.int32),
        _DIMS,
        (1,),
        mode=lax.GatherScatterMode.PROMISE_IN_BOUNDS,
    )


@functools.partial(
    pl.kernel,
    mesh=_mesh,
    out_type=jax.ShapeDtypeStruct((BATCH, SEQ, HIDDEN), jnp.float32),
    scratch_types=[
        pltpu.VMEM((RPW,), jnp.int32),
        pltpu.VMEM((2, HIDDEN), jnp.float32),
        pltpu.VMEM((NBUF, GR, HIDDEN), jnp.float32),
        pltpu.VMEM_SHARED((NS, 2, GR, HIDDEN), jnp.float32),
        pltpu.SemaphoreType.DMA,
        pltpu.SemaphoreType.DMA,
    ],
)
def _sc_lookup(seg_hbm, table_hbm, out_hbm, idx_v, tab_v, bufs, sh, ssem, spsem):
    wid = lax.axis_index("s") * NC + lax.axis_index("c")
    sid = lax.axis_index("s")
    bi = lax.div(wid, WPB)
    srow = lax.rem(wid, WPB) * RPW
    pltpu.sync_copy(seg_hbm.at[bi].at[pl.ds(srow, RPW)], idx_v)
    pltpu.sync_copy(table_hbm, tab_v)
    out_w = out_hbm.at[bi]
    sh_w = sh.at[sid]

    def wait_direct():
        pltpu.make_async_copy(
            out_w.at[pl.ds(srow, GR)], bufs.at[0], ssem
        ).wait()

    def wait_sp():
        pltpu.make_async_copy(
            out_w.at[pl.ds(srow, GR)], sh_w.at[0], spsem
        ).wait()

    def expand(g, b):
        off = pl.multiple_of(g * GR, GR)
        idx16 = idx_v[pl.ds(off, 16)]
        mults = [_lane_splat(idx16, r).astype(jnp.float32) for r in range(GR)]

        def jbody(j, c, _b=b, _mults=mults):
            jo = pl.multiple_of(j * 16, 16)
            t0 = tab_v.at[0][pl.ds(jo, 16)]
            d = tab_v.at[1][pl.ds(jo, 16)] - t0
            for r in range(GR):
                bufs.at[_b].at[r][pl.ds(jo, 16)] = t0 + _mults[r] * d
            return c

        lax.fori_loop(0, JCH, jbody, 0)
        return off

    def outer(k, carry):
        g0 = k * NBUF
        for b in range(NBUF):
            g = g0 + b
            if b % 2 == 0:
                @pl.when(k > 0)
                def _():
                    wait_direct()

                off = expand(g, b)
                pltpu.async_copy(
                    bufs.at[b], out_w.at[pl.ds(srow + off, GR)], ssem
                )
            else:
                slot = b // 2

                @pl.when(k > 0)
                def _():
                    wait_sp()

                off = expand(g, b)
                pltpu.sync_copy(bufs.at[b], sh_w.at[slot])
                pltpu.async_copy(
                    sh_w.at[slot], out_w.at[pl.ds(srow + off, GR)], spsem
                )
        return carry

    lax.fori_loop(0, SUPER, outer, 0)
    for _ in range(2):
        wait_direct()
        wait_sp()


def kernel(segments, table):
    return _sc_lookup(segments.astype(jnp.int32), table)


# speedup vs baseline: 1.8251x; 1.3269x over previous
import functools

import jax
import jax.numpy as jnp
from jax import lax
from jax.experimental import pallas as pl
from jax.experimental.pallas import tpu as pltpu
from jax.experimental.pallas import tpu_sc as plsc

HIDDEN = 1024
BATCH = 4
SEQ = 8192
ROWS = BATCH * SEQ
NC, NS = 2, 16
NW = NC * NS
GR = 16
NBUF = 2
JCH = HIDDEN // 16

SC_BATCH = 2
TC_ROWS = (BATCH - SC_BATCH) * SEQ
TC_BLOCK = 2048

_mesh = plsc.VectorSubcoreMesh(core_axis_name="c", subcore_axis_name="s")

_DIMS = lax.GatherDimensionNumbers(
    offset_dims=(), collapsed_slice_dims=(0,), start_index_map=(0,)
)


def _lane_splat(vec, lane):
    return lax.gather(
        vec,
        jnp.full((16, 1), lane, jnp.int32),
        _DIMS,
        (1,),
        mode=lax.GatherScatterMode.PROMISE_IN_BOUNDS,
    )


RPW = SC_BATCH * SEQ // NW
WPB = SEQ // RPW
GROUPS = RPW // GR


@functools.partial(
    pl.kernel,
    mesh=_mesh,
    out_type=jax.ShapeDtypeStruct((BATCH, SEQ, HIDDEN), jnp.float32),
    scratch_types=[
        pltpu.VMEM((RPW,), jnp.int32),
        pltpu.VMEM((2, HIDDEN), jnp.float32),
        pltpu.VMEM((NBUF, GR, HIDDEN), jnp.float32),
        pltpu.SemaphoreType.DMA,
    ],
)
def _sc_part(seg_hbm, table_hbm, out_hbm, idx_v, tab_v, bufs, ssem):
    wid = lax.axis_index("s") * NC + lax.axis_index("c")
    bi = lax.div(wid, WPB)
    srow = lax.rem(wid, WPB) * RPW
    pltpu.sync_copy(seg_hbm.at[bi].at[pl.ds(srow, RPW)], idx_v)
    pltpu.sync_copy(table_hbm, tab_v)
    out_w = out_hbm.at[bi]

    def wait_one_scatter():
        pltpu.make_async_copy(
            out_w.at[pl.ds(srow, GR)], bufs.at[0], ssem
        ).wait()

    def outer(o, carry):
        for b in range(NBUF):
            g = o * NBUF + b
            off = pl.multiple_of(g * GR, GR)
            idx16 = idx_v[pl.ds(off, 16)]
            mults = [
                _lane_splat(idx16, r).astype(jnp.float32) for r in range(GR)
            ]

            @pl.when(o > 0)
            def _():
                wait_one_scatter()

            def jbody(j, c, _b=b, _mults=mults):
                jo = pl.multiple_of(j * 16, 16)
                t0 = tab_v.at[0][pl.ds(jo, 16)]
                d = tab_v.at[1][pl.ds(jo, 16)] - t0
                for r in range(GR):
                    bufs.at[_b].at[r][pl.ds(jo, 16)] = t0 + _mults[r] * d
                return c

            lax.fori_loop(0, JCH, jbody, 0)
            pltpu.async_copy(
                bufs.at[b], out_w.at[pl.ds(srow + off, GR)], ssem
            )
        return carry

    lax.fori_loop(0, GROUPS // NBUF, outer, 0)
    for _ in range(NBUF):
        wait_one_scatter()


def _tc_body(seg_ref, tab_ref, alias_ref, out_ref):
    del alias_ref
    seg = seg_ref[...]
    t0 = tab_ref[0:1, :]
    t1 = tab_ref[1:2, :]
    out_ref[...] = jnp.where(seg == 0, t0, t1)


_SC_BLOCKS = SC_BATCH * SEQ // TC_BLOCK


def _tc_fill(seg_tc, table, partial):
    seg2 = seg_tc.reshape(TC_ROWS, 1)
    return pl.pallas_call(
        _tc_body,
        grid=(TC_ROWS // TC_BLOCK,),
        in_specs=[
            pl.BlockSpec((TC_BLOCK, 1), lambda i: (i, 0)),
            pl.BlockSpec((2, HIDDEN), lambda i: (0, 0)),
            pl.BlockSpec(memory_space=pl.ANY),
        ],
        out_specs=pl.BlockSpec(
            (TC_BLOCK, HIDDEN), lambda i: (i + _SC_BLOCKS, 0)
        ),
        out_shape=jax.ShapeDtypeStruct((ROWS, HIDDEN), jnp.float32),
        input_output_aliases={2: 0},
    )(seg2, table, partial)


def kernel(segments, table):
    seg = segments.astype(jnp.int32)
    part = _sc_part(seg, table)
    out = _tc_fill(seg[SC_BATCH:], table, part.reshape(ROWS, HIDDEN))
    return out.reshape(BATCH, SEQ, HIDDEN)


# speedup vs baseline: 1.9326x; 1.0589x over previous
import functools

import jax
import jax.numpy as jnp
from jax import lax
from jax.experimental import pallas as pl
from jax.experimental.pallas import tpu as pltpu
from jax.experimental.pallas import tpu_sc as plsc

HIDDEN = 1024
BATCH = 4
SEQ = 8192
ROWS = BATCH * SEQ
NC, NS = 2, 16
NW = NC * NS
GR = 16
NBUF = 2
JCH = HIDDEN // 16

SC_BATCH = 2
TC_ROWS = (BATCH - SC_BATCH) * SEQ
TC_BLOCK = 2048

_mesh = plsc.VectorSubcoreMesh(core_axis_name="c", subcore_axis_name="s")

_DIMS = lax.GatherDimensionNumbers(
    offset_dims=(), collapsed_slice_dims=(0,), start_index_map=(0,)
)


def _lane_splat(vec, lane):
    return lax.gather(
        vec,
        jnp.full((16, 1), lane, jnp.int32),
        _DIMS,
        (1,),
        mode=lax.GatherScatterMode.PROMISE_IN_BOUNDS,
    )


RPW = SC_BATCH * SEQ // NW
WPB = SEQ // RPW
GROUPS = RPW // GR


@functools.partial(
    pl.kernel,
    mesh=_mesh,
    out_type=jax.ShapeDtypeStruct((BATCH, SEQ, HIDDEN), jnp.float32),
    scratch_types=[
        pltpu.VMEM((RPW,), jnp.int32),
        pltpu.VMEM((2, HIDDEN), jnp.float32),
        pltpu.VMEM((NBUF, GR, HIDDEN), jnp.float32),
        pltpu.SemaphoreType.DMA,
    ],
)
def _sc_part(seg_hbm, table_hbm, out_hbm, idx_v, tab_v, bufs, ssem):
    wid = lax.axis_index("s") * NC + lax.axis_index("c")
    bi = lax.div(wid, WPB)
    srow = lax.rem(wid, WPB) * RPW
    pltpu.sync_copy(seg_hbm.at[bi].at[pl.ds(srow, RPW)], idx_v)
    pltpu.sync_copy(table_hbm, tab_v)
    out_w = out_hbm.at[bi]

    def wait_one_scatter():
        pltpu.make_async_copy(
            out_w.at[pl.ds(srow, GR)], bufs.at[0], ssem
        ).wait()

    def outer(o, carry):
        for b in range(NBUF):
            g = o * NBUF + b
            off = pl.multiple_of(g * GR, GR)
            idx16 = idx_v[pl.ds(off, 16)]
            mults = [
                _lane_splat(idx16, r).astype(jnp.float32) for r in range(GR)
            ]

            @pl.when(o > 0)
            def _():
                wait_one_scatter()

            def jbody(j, c, _b=b, _mults=mults):
                jo = pl.multiple_of(j * 16, 16)
                t0 = tab_v.at[0][pl.ds(jo, 16)]
                d = tab_v.at[1][pl.ds(jo, 16)] - t0
                for r in range(GR):
                    bufs.at[_b].at[r][pl.ds(jo, 16)] = t0 + _mults[r] * d
                return c

            lax.fori_loop(0, JCH, jbody, 0)
            pltpu.async_copy(
                bufs.at[b], out_w.at[pl.ds(srow + off, GR)], ssem
            )
        return carry

    lax.fori_loop(0, GROUPS // NBUF, outer, 0)
    for _ in range(NBUF):
        wait_one_scatter()


def _tc_body(seg_ref, tab_ref, alias_ref, out_ref):
    del alias_ref
    b = pl.program_id(0)
    seg = seg_ref[pl.ds(b + SC_BATCH, 1), :]
    segT = jnp.transpose(seg)
    t0 = tab_ref[0:1, :]
    t1 = tab_ref[1:2, :]
    out_ref[...] = jnp.where(segT == 0, t0, t1)


_SC_BLOCKS = SC_BATCH * SEQ // TC_BLOCK
_CPB = SEQ // TC_BLOCK


def _tc_fill(seg, table, partial):
    return pl.pallas_call(
        _tc_body,
        grid=(BATCH - SC_BATCH, _CPB),
        in_specs=[
            pl.BlockSpec((BATCH, TC_BLOCK), lambda b, c: (0, c)),
            pl.BlockSpec((2, HIDDEN), lambda b, c: (0, 0)),
            pl.BlockSpec(memory_space=pl.ANY),
        ],
        out_specs=pl.BlockSpec(
            (TC_BLOCK, HIDDEN),
            lambda b, c: (_SC_BLOCKS + b * _CPB + c, 0),
        ),
        out_shape=jax.ShapeDtypeStruct((ROWS, HIDDEN), jnp.float32),
        input_output_aliases={2: 0},
    )(seg, table, partial)


def kernel(segments, table):
    seg = segments.astype(jnp.int32)
    part = _sc_part(seg, table)
    out = _tc_fill(seg, table, part.reshape(ROWS, HIDDEN))
    return out.reshape(BATCH, SEQ, HIDDEN)


# speedup vs baseline: 2.1125x; 1.0931x over previous
import functools

import jax
import jax.numpy as jnp
from jax import lax
from jax.experimental import pallas as pl
from jax.experimental.pallas import tpu as pltpu
from jax.experimental.pallas import tpu_sc as plsc

HIDDEN = 1024
BATCH = 4
SEQ = 8192
ROWS = BATCH * SEQ
NC, NS = 2, 16
NW = NC * NS
GR = 16
NBUF = 2
JCH = HIDDEN // 16

SC_BATCH = 1
TC_ROWS = (BATCH - SC_BATCH) * SEQ
TC_BLOCK = 2048

_mesh = plsc.VectorSubcoreMesh(core_axis_name="c", subcore_axis_name="s")

_DIMS = lax.GatherDimensionNumbers(
    offset_dims=(), collapsed_slice_dims=(0,), start_index_map=(0,)
)


def _lane_splat(vec, lane):
    return lax.gather(
        vec,
        jnp.full((16, 1), lane, jnp.int32),
        _DIMS,
        (1,),
        mode=lax.GatherScatterMode.PROMISE_IN_BOUNDS,
    )


RPW = SC_BATCH * SEQ // NW
WPB = SEQ // RPW
GROUPS = RPW // GR


@functools.partial(
    pl.kernel,
    mesh=_mesh,
    out_type=jax.ShapeDtypeStruct((BATCH, SEQ, HIDDEN), jnp.float32),
    scratch_types=[
        pltpu.VMEM((RPW,), jnp.int32),
        pltpu.VMEM((2, HIDDEN), jnp.float32),
        pltpu.VMEM((NBUF, GR, HIDDEN), jnp.float32),
        pltpu.SemaphoreType.DMA,
    ],
)
def _sc_part(seg_hbm, table_hbm, out_hbm, idx_v, tab_v, bufs, ssem):
    wid = lax.axis_index("s") * NC + lax.axis_index("c")
    bi = lax.div(wid, WPB)
    srow = lax.rem(wid, WPB) * RPW
    pltpu.sync_copy(seg_hbm.at[bi].at[pl.ds(srow, RPW)], idx_v)
    pltpu.sync_copy(table_hbm, tab_v)
    out_w = out_hbm.at[bi]

    def wait_one_scatter():
        pltpu.make_async_copy(
            out_w.at[pl.ds(srow, GR)], bufs.at[0], ssem
        ).wait()

    def outer(o, carry):
        for b in range(NBUF):
            g = o * NBUF + b
            off = pl.multiple_of(g * GR, GR)
            idx16 = idx_v[pl.ds(off, 16)]
            mults = [
                _lane_splat(idx16, r).astype(jnp.float32) for r in range(GR)
            ]

            @pl.when(o > 0)
            def _():
                wait_one_scatter()

            def jbody(j, c, _b=b, _mults=mults):
                jo = pl.multiple_of(j * 16, 16)
                t0 = tab_v.at[0][pl.ds(jo, 16)]
                d = tab_v.at[1][pl.ds(jo, 16)] - t0
                for r in range(GR):
                    bufs.at[_b].at[r][pl.ds(jo, 16)] = t0 + _mults[r] * d
                return c

            lax.fori_loop(0, JCH, jbody, 0)
            pltpu.async_copy(
                bufs.at[b], out_w.at[pl.ds(srow + off, GR)], ssem
            )
        return carry

    lax.fori_loop(0, GROUPS // NBUF, outer, 0)
    for _ in range(NBUF):
        wait_one_scatter()


def _tc_body(seg_ref, tab_ref, alias_ref, out_ref):
    del alias_ref
    b = pl.program_id(0)
    seg = seg_ref[pl.ds(b + SC_BATCH, 1), :]
    segT = jnp.transpose(seg)
    t0 = tab_ref[0:1, :]
    t1 = tab_ref[1:2, :]
    out_ref[...] = jnp.where(segT == 0, t0, t1)


_SC_BLOCKS = SC_BATCH * SEQ // TC_BLOCK
_CPB = SEQ // TC_BLOCK


def _tc_fill(seg, table, partial):
    return pl.pallas_call(
        _tc_body,
        grid=(BATCH - SC_BATCH, _CPB),
        in_specs=[
            pl.BlockSpec((BATCH, TC_BLOCK), lambda b, c: (0, c)),
            pl.BlockSpec((2, HIDDEN), lambda b, c: (0, 0)),
            pl.BlockSpec(memory_space=pl.ANY),
        ],
        out_specs=pl.BlockSpec(
            (TC_BLOCK, HIDDEN),
            lambda b, c: (_SC_BLOCKS + b * _CPB + c, 0),
        ),
        out_shape=jax.ShapeDtypeStruct((ROWS, HIDDEN), jnp.float32),
        input_output_aliases={2: 0},
    )(seg, table, partial)


def kernel(segments, table):
    seg = segments.astype(jnp.int32)
    part = _sc_part(seg, table)
    out = _tc_fill(seg, table, part.reshape(ROWS, HIDDEN))
    return out.reshape(BATCH, SEQ, HIDDEN)
